# baseline (speedup 1.0000x reference)
import functools

import jax
import jax.numpy as jnp
from jax import lax
from jax.experimental import pallas as pl
from jax.experimental.pallas import tpu as pltpu
from jax.experimental.pallas import tpu_sc as plsc


def _sc_gather_add(table, idx3, D):
    NW, NCH, CH = idx3.shape
    V, DP = table.shape
    NC = 2

    mesh = plsc.VectorSubcoreMesh(core_axis_name="c", subcore_axis_name="s")

    NBUF = 2
    assert NCH % NBUF == 0
    NGRP = NCH // NBUF
    NQ = D // 16

    @functools.partial(
        pl.kernel,
        out_type=jax.ShapeDtypeStruct((NW, NCH, CH, D), table.dtype),
        mesh=mesh,
        scratch_types=[
            pltpu.VMEM((NCH, CH), jnp.int32),
            pltpu.VMEM((NBUF, CH, DP), jnp.float32),
            pltpu.VMEM((NBUF, CH, D), jnp.float32),
            pltpu.SemaphoreType.DMA,
            pltpu.SemaphoreType.DMA,
        ],
    )
    def gather_kernel(tab_hbm, idx_hbm, out_hbm, idx_v, rows_v, out_v, gsem, ssem):
        wid = lax.axis_index("s") * NC + lax.axis_index("c")
        pltpu.sync_copy(idx_hbm.at[wid], idx_v)

        def gather(c, b):
            return pltpu.make_async_copy(
                tab_hbm.at[idx_v.at[c]], rows_v.at[b], gsem)

        def store(c, b):
            return pltpu.make_async_copy(
                out_v.at[b], out_hbm.at[wid, c], ssem)

        for b in range(NBUF):
            gather(b, b).start()

        def grp(g, carry):
            for b in range(NBUF):
                c = g * NBUF + b
                gather(c, b).wait()

                @pl.when(g > 0)
                def _():
                    store(c - NBUF, b).wait()

                @plsc.parallel_loop(0, CH, step=1, unroll=8)
                def _(i):
                    for q in range(NQ):
                        out_v[b, i, pl.ds(q * 16, 16)] = (
                            rows_v[b, i, pl.ds(q * 16, 16)]
                            + rows_v[b, i, pl.ds(D + q * 16, 16)])

                store(c, b).start()

                @pl.when(g < NGRP - 1)
                def _():
                    gather(c + NBUF, b).start()
            return carry

        lax.fori_loop(0, NGRP, grp, 0)
        for b in range(NBUF):
            store((NGRP - 1) * NBUF + b, b).wait()

    return gather_kernel(table, idx3)


def kernel(x, token_embed, pos_embed):
    B, S = x.shape
    V, D = token_embed.shape
    combined = jnp.concatenate([token_embed, pos_embed], axis=1)

    NW = 32
    CH = 128
    total = B * S
    assert total % (NW * CH) == 0
    NCH = total // (NW * CH)
    idx3 = x.reshape(NW, NCH, CH).astype(jnp.int32)
    out = _sc_gather_add(combined, idx3, D)
    return out.reshape(B, S, D)
